# baseline (speedup 1.0000x reference)
import functools

import jax
import jax.numpy as jnp
from jax import lax
from jax.experimental import pallas as pl
from jax.experimental.pallas import tpu as pltpu
from jax.experimental.pallas import tpu_sc as plsc

B, L, C = 64, 2048, 8
K, V, ED = 26, 100000, 16
CONT = 8
F = 1 + CONT + K * ED
W = 512
EPS = 1e-5
LN = 16
NC = 2
NW = 32
BPW = B // NW
CH = 128
NCH = L // CH
LT = L // 128

_TMAP = {(0, 0): None}
for _u in range(1, 8):
    _TMAP[(0, _u)] = _u - 1
for _c, _base in ((1, 7), (2, 15), (3, 23)):
    for _u in range(8):
        _t = _base + _u
        _TMAP[(_c, _u)] = _t if _t < K else None


def _rsqrt16(v):
    bits = lax.bitcast_convert_type(v, jnp.int32)
    i = jnp.int32(0x5F3759DF) - lax.shift_right_logical(bits, 1)
    y = lax.bitcast_convert_type(i, jnp.float32)
    for _ in range(3):
        y = y * (1.5 - 0.5 * v * y * y)
    return y


_mesh = plsc.VectorSubcoreMesh(core_axis_name="c", subcore_axis_name="s")


@functools.partial(
    pl.kernel,
    out_type=jax.ShapeDtypeStruct((B * L // 8, 4, 64, 16), jnp.float32),
    mesh=_mesh,
    compiler_params=pltpu.CompilerParams(
        use_tc_tiling_on_sc=False, needs_layout_passes=False),
    scratch_types=[
        pltpu.VMEM((L,), jnp.float32),
        pltpu.VMEM((CONT, CH), jnp.float32),
        pltpu.VMEM((K, CH), jnp.int32),
        pltpu.VMEM((4, CH * 8), jnp.int32),
        pltpu.VMEM((4, 16, 1, 64, 16), jnp.float32),
        pltpu.VMEM((LN,), jnp.float32),
        pltpu.VMEM((LN,), jnp.float32),
        pltpu.SemaphoreType.DMA,
    ],
)
def _titan_sc(xp, cp, catp, rw, rb, tab, out,
              xb, cb, ixt, ixs, gbuf, rwv, rbv, sem):
    wid = lax.axis_index("s") * NC + lax.axis_index("c")
    iota = lax.iota(jnp.int32, LN)
    pltpu.sync_copy(rw, rwv)
    pltpu.sync_copy(rb, rbv)
    w16 = rwv[...]
    b16 = rbv[...]
    zi = jnp.zeros((LN,), jnp.int32)

    def batch_body(bi, _):
        b = wid * BPW + bi
        b8 = b // 8
        br = b - b8 * 8

        def xload(lt, _x):
            pltpu.sync_copy(xp.at[b, lt, 0], xb.at[pl.ds(lt * 128, 128)])
            return _x
        lax.fori_loop(0, LT, xload, None)

        s = jnp.zeros((LN,), jnp.float32)
        s2 = jnp.zeros((LN,), jnp.float32)
        for v in range(L // LN):
            xv = xb[pl.ds(v * LN, LN)]
            s = s + xv
            s2 = s2 + xv * xv
        inv_n = jnp.float32(1.0 / L)
        mu = jnp.sum(s) * inv_n
        var = jnp.sum(s2) * inv_n - mu * mu
        inv = _rsqrt16(jnp.full((LN,), var + EPS, jnp.float32))
        a = inv * w16
        c0 = b16 - mu * a

        def chunk_body(ci, _c):
            l0 = ci * CH
            r80 = b * (L // 8) + ci * 16
            pltpu.sync_copy(catp.at[:, b8, ci, br], ixt)
            for c in range(4):
                cv = jnp.full((LN,), c, jnp.int32)
                for u in range(8):
                    t = _TMAP[(c, u)]
                    for v in range(CH // LN):
                        p16 = iota + v * LN
                        dst = p16 * 8 + u
                        if t is None:
                            val = zi
                        else:
                            val = (ixt[t, pl.ds(v * LN, LN)]
                                   + jnp.full((LN,), t * V, jnp.int32))
                        plsc.store_scatter(ixs, [cv, dst], val)
            cps = []
            for c in range(4):
                for s16 in range(16):
                    cps.append(pltpu.async_copy(
                        tab.at[ixs.at[c, pl.ds(s16 * 64, 64)]],
                        gbuf.at[c, s16, 0], sem))
            pltpu.sync_copy(cp.at[b, ci], cb)
            for cp_ in cps[:16]:
                cp_.wait()
            for v in range(CH // LN):
                p16 = iota + v * LN
                h1 = lax.shift_right_logical(p16, 3)
                h3 = (p16 - h1 * 8) * 8
                xv = xb[pl.ds(l0 + v * LN, LN)]
                plsc.store_scatter(
                    gbuf, [zi, h1, zi, h3, jnp.full((LN,), 7, jnp.int32)],
                    xv * a + c0)
                for q in range(CONT):
                    plsc.store_scatter(
                        gbuf,
                        [zi, h1, zi, h3, jnp.full((LN,), 8 + q, jnp.int32)],
                        cb[q, pl.ds(v * LN, LN)])
            for c in range(4):
                if c:
                    for cp_ in cps[c * 16:(c + 1) * 16]:
                        cp_.wait()
                pltpu.sync_copy(
                    gbuf.at[c],
                    out.at[pl.ds(r80, 16), pl.ds(c, 1)])
            return _c

        return lax.fori_loop(0, NCH, chunk_body, _)

    lax.fori_loop(0, BPW, batch_body, None)


def _depad_body(g_hbm, o_ref, vbuf, sem):
    b8 = pl.program_id(0)
    lt = pl.program_id(1)
    for j in range(8):
        q0 = (b8 * 8 + j) * (L // 8) + lt * 16
        pltpu.make_async_copy(
            g_hbm.at[pl.ds(q0, 16)], vbuf.at[j], sem).start()
    for j in range(8):
        q0 = (b8 * 8 + j) * (L // 8) + lt * 16
        pltpu.make_async_copy(
            g_hbm.at[pl.ds(q0, 16)], vbuf.at[j], sem).wait()
    bounds = ((7, 0, 121), (0, 121, 249), (0, 249, 377), (0, 377, 425))
    for j in range(8):
        for c in range(4):
            t = jnp.transpose(vbuf[j, :, c], (2, 0, 1)).reshape(128, 128)
            m0, f0, f1 = bounds[c]
            o_ref[f0:f1, 0, 0, j, :] = t[m0:m0 + (f1 - f0)]


_depad_tc = pl.pallas_call(
    _depad_body,
    grid=(B // 8, LT),
    in_specs=[pl.BlockSpec(memory_space=pl.ANY)],
    out_specs=pl.BlockSpec((F, 1, 1, 8, 128), lambda b8, lt: (0, b8, lt, 0, 0)),
    out_shape=jax.ShapeDtypeStruct((F, B // 8, LT, 8, 128), jnp.float32),
    scratch_shapes=[pltpu.VMEM((8, 16, 4, 8, 128), jnp.float32),
                    pltpu.SemaphoreType.DMA],
)


def kernel(x, past_exo_cont, past_exo_cat, revin_weight, revin_bias,
           emb_tables):
    xpv = x.transpose(0, 2, 1).reshape(B, C, LT, 128).transpose(0, 2, 1, 3)
    cpv = (past_exo_cont.transpose(0, 2, 1)
           .reshape(B, CONT, LT, 128).transpose(0, 2, 1, 3))
    catp = (past_exo_cat.astype(jnp.int32).transpose(2, 0, 1)
            .reshape(K, B // 8, 8, LT, 128).transpose(0, 1, 3, 2, 4))
    tab = emb_tables.reshape(K * V, ED)
    rw = jnp.broadcast_to(revin_weight.astype(jnp.float32), (LN,))
    rb = jnp.broadcast_to(revin_bias.astype(jnp.float32), (LN,))
    slabs = _titan_sc(xpv, cpv, catp, rw, rb, tab)
    gin = slabs.reshape(B * L // 8, 4, 8, 128)
    tcout = _depad_tc(gin)
    return tcout.transpose(1, 3, 2, 4, 0).reshape(B, L, F)


# speedup vs baseline: 7.1274x; 7.1274x over previous
import functools

import jax
import jax.numpy as jnp
from jax import lax
from jax.experimental import pallas as pl
from jax.experimental.pallas import tpu as pltpu
from jax.experimental.pallas import tpu_sc as plsc

B, L, C = 64, 2048, 8
K, V, ED = 26, 100000, 16
CONT = 8
F = 1 + CONT + K * ED
EPS = 1e-5
LN = 16
NC = 2
NW = 32
BPW = B // NW
CH = 128
NCH = L // CH
LT = L // 128


def _rsqrt16(v):
    bits = lax.bitcast_convert_type(v, jnp.int32)
    i = jnp.int32(0x5F3759DF) - lax.shift_right_logical(bits, 1)
    y = lax.bitcast_convert_type(i, jnp.float32)
    for _ in range(3):
        y = y * (1.5 - 0.5 * v * y * y)
    return y


_mesh = plsc.VectorSubcoreMesh(core_axis_name="c", subcore_axis_name="s")


@functools.partial(
    pl.kernel,
    out_type=jax.ShapeDtypeStruct((B * L, 7 + F), jnp.float32),
    mesh=_mesh,
    compiler_params=pltpu.CompilerParams(
        use_tc_tiling_on_sc=False, needs_layout_passes=False),
    scratch_types=[
        pltpu.VMEM((L,), jnp.float32),
        pltpu.VMEM((CONT, CH), jnp.float32),
        pltpu.VMEM((K, CH), jnp.int32),
        pltpu.VMEM((K, CH, ED), jnp.float32),
        pltpu.VMEM((CH, 16), jnp.float32),
        pltpu.VMEM((LN,), jnp.float32),
        pltpu.VMEM((LN,), jnp.float32),
        pltpu.SemaphoreType.DMA,
    ],
)
def _titan_sc(xp, cp, catp, rw, rb, tab, out,
              xb, cb, ixt, emb, head, rwv, rbv, sem):
    wid = lax.axis_index("s") * NC + lax.axis_index("c")
    iota = lax.iota(jnp.int32, LN)
    pltpu.sync_copy(rw, rwv)
    pltpu.sync_copy(rb, rbv)
    w16 = rwv[...]
    b16 = rbv[...]

    def batch_body(bi, _):
        b = wid * BPW + bi
        b8 = b // 8
        br = b - b8 * 8
        def xload(lt, _x):
            pltpu.sync_copy(xp.at[b, lt, 0], xb.at[pl.ds(lt * 128, 128)])
            return _x
        lax.fori_loop(0, LT, xload, None)

        s = jnp.zeros((LN,), jnp.float32)
        s2 = jnp.zeros((LN,), jnp.float32)
        for v in range(L // LN):
            xv = xb[pl.ds(v * LN, LN)]
            s = s + xv
            s2 = s2 + xv * xv
        inv_n = jnp.float32(1.0 / L)
        mu = jnp.sum(s) * inv_n
        var = jnp.sum(s2) * inv_n - mu * mu
        inv = _rsqrt16(jnp.full((LN,), var + EPS, jnp.float32))
        a = inv * w16
        c0 = b16 - mu * a

        def chunk_body(ci, _c):
            l0 = ci * CH
            row0 = b * L + l0
            pltpu.sync_copy(catp.at[:, b8, ci, br], ixt)
            for i in range(K):
                ofs = jnp.full((LN,), i * V, jnp.int32)
                for v in range(CH // LN):
                    sl = pl.ds(v * LN, LN)
                    ixt[i, sl] = ixt[i, sl] + ofs
            cps = []
            for i in range(K):
                cps.append(pltpu.async_copy(tab.at[ixt.at[i]], emb.at[i], sem))
            pltpu.sync_copy(cp.at[b, ci], cb)
            s7 = jnp.full((LN,), 7, jnp.int32)
            for v in range(CH // LN):
                p16 = iota + v * LN
                xv = xb[pl.ds(l0 + v * LN, LN)]
                plsc.store_scatter(head, [p16, s7], xv * a + c0)
            for q in range(CONT):
                cq = jnp.full((LN,), q + 8, jnp.int32)
                for v in range(CH // LN):
                    cv = cb[q, pl.ds(v * LN, LN)]
                    plsc.store_scatter(head, [iota + v * LN, cq], cv)
            pltpu.sync_copy(head, out.at[pl.ds(row0, CH), pl.ds(0, 16)])
            for i in range(K):
                cps[i].wait()
                pltpu.sync_copy(
                    emb.at[i],
                    out.at[pl.ds(row0, CH), pl.ds(16 + i * ED, ED)])
            return _c

        return lax.fori_loop(0, NCH, chunk_body, _)

    lax.fori_loop(0, BPW, batch_body, None)


def kernel(x, past_exo_cont, past_exo_cat, revin_weight, revin_bias,
           emb_tables):
    xpv = x.transpose(0, 2, 1).reshape(B, C, LT, 128).transpose(0, 2, 1, 3)
    cpv = (past_exo_cont.transpose(0, 2, 1)
           .reshape(B, CONT, LT, 128).transpose(0, 2, 1, 3))
    catp = (past_exo_cat.astype(jnp.int32).transpose(2, 0, 1)
            .reshape(K, B // 8, 8, LT, 128).transpose(0, 1, 3, 2, 4))
    tab = emb_tables.reshape(K * V, ED)
    rw = jnp.broadcast_to(revin_weight.astype(jnp.float32), (LN,))
    rb = jnp.broadcast_to(revin_bias.astype(jnp.float32), (LN,))
    padded = _titan_sc(xpv, cpv, catp, rw, rb, tab)
    return padded.reshape(B, L, 7 + F)[:, :, 7:]
